# Initial kernel scaffold; baseline (speedup 1.0000x reference)
import jax
import jax.numpy as jnp
from jax.experimental import pallas as pl


def kernel(x, W_fc, W_proj, W_router):
    raise NotImplementedError("write your pallas kernel here")


# speedup vs baseline: 1.7386x; 1.7386x over previous
import functools

import jax
import jax.numpy as jnp
from jax import lax
from jax.experimental import pallas as pl
from jax.experimental.pallas import tpu as pltpu
from jax.experimental.pallas import tpu_sc as plsc

DIM = 2048
E = 8
HID = 1024
BM = 256
CHUNK = 32


def _router_body(x_ref, w_ref, out_ref):
    logits = lax.dot_general(
        x_ref[...], w_ref[...], (((1,), (1,)), ((), ())),
        preferred_element_type=jnp.float32)
    out_ref[...] = jnp.argmax(logits, axis=-1).astype(jnp.int32)[None, None, :]


def _router(xf, W_router):
    T = xf.shape[0]
    RB = 1024
    return pl.pallas_call(
        _router_body,
        grid=(T // RB,),
        in_specs=[
            pl.BlockSpec((RB, DIM), lambda i: (i, 0)),
            pl.BlockSpec((E, DIM), lambda i: (0, 0)),
        ],
        out_specs=pl.BlockSpec((1, 1, RB), lambda i: (i, 0, 0)),
        out_shape=jax.ShapeDtypeStruct((T // RB, 1, RB), jnp.int32),
    )(xf, W_router)


def _mlp_body(be_ref, na_ref, x_ref, fc_ref, proj_ref, out_ref):
    b = pl.program_id(0)

    @pl.when(b < na_ref[0])
    def _():
        h = lax.dot_general(
            x_ref[...], fc_ref[0], (((1,), (1,)), ((), ())),
            preferred_element_type=jnp.float32)
        h = jnp.where(h >= 0, h, 0.5 * h)
        h = h * h
        out_ref[...] = lax.dot_general(
            h, proj_ref[0], (((1,), (1,)), ((), ())),
            preferred_element_type=jnp.float32)


def _grouped_mlp(xg, W_fc, W_proj, block_expert, n_active, nb):
    grid_spec = pltpu.PrefetchScalarGridSpec(
        num_scalar_prefetch=2,
        grid=(nb,),
        in_specs=[
            pl.BlockSpec((BM, DIM), lambda b, be, na: (b, 0)),
            pl.BlockSpec((1, HID, DIM), lambda b, be, na: (be[b], 0, 0)),
            pl.BlockSpec((1, DIM, HID), lambda b, be, na: (be[b], 0, 0)),
        ],
        out_specs=pl.BlockSpec((BM, DIM), lambda b, be, na: (b, 0)),
    )
    return pl.pallas_call(
        _mlp_body,
        grid_spec=grid_spec,
        out_shape=jax.ShapeDtypeStruct((xg.shape[0], DIM), jnp.float32),
    )(block_expert, n_active, xg, W_fc, W_proj)


@functools.cache
def _make_sc_gather(B, D):
    info = plsc.get_sparse_core_info()
    NC, NS = info.num_cores, info.num_subcores
    NW = NC * NS
    b_per_w = B // NW
    n_chunks = b_per_w // CHUNK
    mesh = plsc.VectorSubcoreMesh(core_axis_name="c", subcore_axis_name="s")

    @functools.partial(
        pl.kernel,
        mesh=mesh,
        out_type=jax.ShapeDtypeStruct((B, D), jnp.float32),
        scratch_types=[
            pltpu.VMEM((CHUNK,), jnp.int32),
            pltpu.VMEM((CHUNK, D), jnp.float32),
            pltpu.SemaphoreType.DMA,
        ],
    )
    def k(table_hbm, idx_hbm, out_hbm, idx_v, rows_v, sem):
        wid = lax.axis_index("s") * NC + lax.axis_index("c")
        base = wid * b_per_w

        def body(c, _):
            off = pl.multiple_of(base + c * CHUNK, CHUNK)
            pltpu.sync_copy(idx_hbm.at[pl.ds(off, CHUNK)], idx_v)
            pltpu.async_copy(table_hbm.at[idx_v], rows_v, sem).wait()
            pltpu.sync_copy(rows_v, out_hbm.at[pl.ds(off, CHUNK)])
            return 0

        lax.fori_loop(0, n_chunks, body, 0)

    return k


def kernel(x, W_fc, W_proj, W_router):
    bsz, seqlen, dim = x.shape
    T = bsz * seqlen
    xf = x.reshape(T, dim)
    pad_m = T + E * BM
    nb = pad_m // BM

    idx = _router(xf, W_router).reshape(T)

    oh = (idx[:, None] == jnp.arange(E, dtype=jnp.int32)[None, :]).astype(jnp.int32)
    counts = jnp.sum(oh, axis=0)
    rank = jnp.take_along_axis(jnp.cumsum(oh, axis=0) - 1, idx[:, None], axis=1)[:, 0]
    pc = ((counts + BM - 1) // BM) * BM
    ends = jnp.cumsum(pc)
    dest = (ends - pc)[idx] + rank
    src = jnp.zeros((pad_m,), jnp.int32).at[dest].set(
        jnp.arange(T, dtype=jnp.int32))
    n_active = (ends[-1] // BM).astype(jnp.int32).reshape(1)
    block_expert = jnp.clip(
        jnp.searchsorted(ends, jnp.arange(nb, dtype=jnp.int32) * BM, side="right"),
        0, E - 1).astype(jnp.int32)

    xg = _make_sc_gather(pad_m, dim)(xf, src)

    yg = _grouped_mlp(xg, W_fc, W_proj, block_expert, n_active, nb)

    out = _make_sc_gather(T, dim)(yg, dest)

    return out.reshape(bsz, seqlen, dim)


# speedup vs baseline: 2.7484x; 1.5808x over previous
import functools

import jax
import jax.numpy as jnp
from jax import lax
from jax.experimental import pallas as pl
from jax.experimental.pallas import tpu as pltpu
from jax.experimental.pallas import tpu_sc as plsc

DIM = 2048
E = 8
HID = 1024
BM = 256
CHUNK = 16
NBUF = 3


def _router_body(x_ref, w_ref, out_ref):
    logits = lax.dot_general(
        x_ref[...], w_ref[...], (((1,), (1,)), ((), ())),
        preferred_element_type=jnp.float32)
    out_ref[...] = jnp.argmax(logits, axis=-1).astype(jnp.int32)[None, None, :]


def _router(xf, W_router):
    T = xf.shape[0]
    RB = 1024
    return pl.pallas_call(
        _router_body,
        grid=(T // RB,),
        in_specs=[
            pl.BlockSpec((RB, DIM), lambda i: (i, 0)),
            pl.BlockSpec((E, DIM), lambda i: (0, 0)),
        ],
        out_specs=pl.BlockSpec((1, 1, RB), lambda i: (i, 0, 0)),
        out_shape=jax.ShapeDtypeStruct((T // RB, 1, RB), jnp.int32),
    )(xf, W_router)


def _mlp_body(be_ref, na_ref, x_ref, fc_ref, proj_ref, out_ref):
    b = pl.program_id(0)

    @pl.when(b < na_ref[0])
    def _():
        h = lax.dot_general(
            x_ref[...], fc_ref[0], (((1,), (1,)), ((), ())),
            preferred_element_type=jnp.float32)
        h = jnp.where(h >= 0, h, 0.5 * h)
        h = h * h
        out_ref[...] = lax.dot_general(
            h, proj_ref[0], (((1,), (1,)), ((), ())),
            preferred_element_type=jnp.float32)


def _grouped_mlp(xg, W_fc, W_proj, block_expert, n_active, nb):
    grid_spec = pltpu.PrefetchScalarGridSpec(
        num_scalar_prefetch=2,
        grid=(nb,),
        in_specs=[
            pl.BlockSpec((BM, DIM), lambda b, be, na: (b, 0)),
            pl.BlockSpec((1, HID, DIM), lambda b, be, na: (be[b], 0, 0)),
            pl.BlockSpec((1, DIM, HID), lambda b, be, na: (be[b], 0, 0)),
        ],
        out_specs=pl.BlockSpec((BM, DIM), lambda b, be, na: (b, 0)),
    )
    return pl.pallas_call(
        _mlp_body,
        grid_spec=grid_spec,
        out_shape=jax.ShapeDtypeStruct((xg.shape[0], DIM), jnp.float32),
    )(block_expert, n_active, xg, W_fc, W_proj)


def _sc_scratch(n, D):
    return (
        [pltpu.VMEM((n, CHUNK), jnp.int32)]
        + [pltpu.VMEM((CHUNK, D), jnp.float32) for _ in range(NBUF)]
        + [pltpu.SemaphoreType.DMA for _ in range(2 * NBUF)]
    )


@functools.cache
def _make_sc_scatter_rows(T, PAD, D):
    info = plsc.get_sparse_core_info()
    NC, NS = info.num_cores, info.num_subcores
    NW = NC * NS
    rows_pw = T // NW
    n = rows_pw // CHUNK
    mesh = plsc.VectorSubcoreMesh(core_axis_name="c", subcore_axis_name="s")

    @functools.partial(
        pl.kernel, mesh=mesh,
        out_type=jax.ShapeDtypeStruct((PAD, D), jnp.float32),
        scratch_types=_sc_scratch(n, D),
    )
    def k(x_hbm, idx2_hbm, out_hbm, idx_all, *rest):
        bufs, sin, sout = rest[:NBUF], rest[NBUF:2 * NBUF], rest[2 * NBUF:]
        wid = lax.axis_index("s") * NC + lax.axis_index("c")
        r0 = wid * rows_pw
        pltpu.sync_copy(idx2_hbm.at[pl.ds(wid * n, n)], idx_all)
        ics = [None] * n
        ocs = [None] * n
        for c in range(min(NBUF, n)):
            off = pl.multiple_of(r0 + c * CHUNK, CHUNK)
            ics[c] = pltpu.async_copy(
                x_hbm.at[pl.ds(off, CHUNK)], bufs[c % NBUF], sin[c % NBUF])
        for c in range(n):
            b = c % NBUF
            ics[c].wait()
            ocs[c] = pltpu.async_copy(bufs[b], out_hbm.at[idx_all.at[c]], sout[b])
            nx = c + NBUF
            if nx < n:
                ocs[c].wait()
                off = pl.multiple_of(r0 + nx * CHUNK, CHUNK)
                ics[nx] = pltpu.async_copy(
                    x_hbm.at[pl.ds(off, CHUNK)], bufs[b], sin[b])
        for c in range(max(n - NBUF, 0), n):
            ocs[c].wait()

    return k


@functools.cache
def _make_sc_gather_rows(T, PAD, D):
    info = plsc.get_sparse_core_info()
    NC, NS = info.num_cores, info.num_subcores
    NW = NC * NS
    rows_pw = T // NW
    n = rows_pw // CHUNK
    mesh = plsc.VectorSubcoreMesh(core_axis_name="c", subcore_axis_name="s")

    @functools.partial(
        pl.kernel, mesh=mesh,
        out_type=jax.ShapeDtypeStruct((T, D), jnp.float32),
        scratch_types=_sc_scratch(n, D),
    )
    def k(tbl_hbm, idx2_hbm, out_hbm, idx_all, *rest):
        bufs, sin, sout = rest[:NBUF], rest[NBUF:2 * NBUF], rest[2 * NBUF:]
        wid = lax.axis_index("s") * NC + lax.axis_index("c")
        r0 = wid * rows_pw
        pltpu.sync_copy(idx2_hbm.at[pl.ds(wid * n, n)], idx_all)
        ics = [None] * n
        ocs = [None] * n
        for c in range(min(NBUF, n)):
            ics[c] = pltpu.async_copy(
                tbl_hbm.at[idx_all.at[c]], bufs[c % NBUF], sin[c % NBUF])
        for c in range(n):
            b = c % NBUF
            ics[c].wait()
            off = pl.multiple_of(r0 + c * CHUNK, CHUNK)
            ocs[c] = pltpu.async_copy(bufs[b], out_hbm.at[pl.ds(off, CHUNK)], sout[b])
            nx = c + NBUF
            if nx < n:
                ocs[c].wait()
                ics[nx] = pltpu.async_copy(
                    tbl_hbm.at[idx_all.at[nx]], bufs[b], sin[b])
        for c in range(max(n - NBUF, 0), n):
            ocs[c].wait()

    return k


def kernel(x, W_fc, W_proj, W_router):
    bsz, seqlen, dim = x.shape
    T = bsz * seqlen
    xf = x.reshape(T, dim)
    pad_m = T + E * BM
    nb = pad_m // BM

    idx = _router(xf, W_router).reshape(T)

    oh = (idx[:, None] == jnp.arange(E, dtype=jnp.int32)[None, :]).astype(jnp.int32)
    counts = jnp.sum(oh, axis=0)
    rank = jnp.take_along_axis(jnp.cumsum(oh, axis=0) - 1, idx[:, None], axis=1)[:, 0]
    pc = ((counts + BM - 1) // BM) * BM
    ends = jnp.cumsum(pc)
    dest = (ends - pc)[idx] + rank
    dest2 = dest.astype(jnp.int32).reshape(T // CHUNK, CHUNK)
    n_active = (ends[-1] // BM).astype(jnp.int32).reshape(1)
    block_expert = jnp.clip(
        jnp.searchsorted(ends, jnp.arange(nb, dtype=jnp.int32) * BM, side="right"),
        0, E - 1).astype(jnp.int32)

    xg = _make_sc_scatter_rows(T, pad_m, dim)(xf, dest2)

    yg = _grouped_mlp(xg, W_fc, W_proj, block_expert, n_active, nb)

    out = _make_sc_gather_rows(T, pad_m, dim)(yg, dest2)

    return out.reshape(bsz, seqlen, dim)
